# baseline (speedup 1.0000x reference)
import functools

import jax
import jax.numpy as jnp
from jax import lax
from jax.experimental import pallas as pl
from jax.experimental.pallas import tpu as pltpu
from jax.experimental.pallas import tpu_sc as plsc

_NUM_EMB = 1000000
_EMB_DIM = 64
_B = 4096 * 50

_SCALE_ROWS = 8000


def _scale_body(w_ref, out_ref, acc_ref):
    i = pl.program_id(0)

    @pl.when(i == 0)
    def _():
        acc_ref[0] = 0.0

    acc_ref[0] = jnp.maximum(acc_ref[0], jnp.max(jnp.abs(w_ref[...])))

    @pl.when(i == pl.num_programs(0) - 1)
    def _():
        out_ref[0] = jnp.maximum(acc_ref[0], 1e-8) / 127.0


_scale_call = pl.pallas_call(
    _scale_body,
    grid=(_NUM_EMB // _SCALE_ROWS,),
    in_specs=[pl.BlockSpec((_SCALE_ROWS, _EMB_DIM), lambda i: (i, 0))],
    out_specs=pl.BlockSpec(memory_space=pltpu.SMEM),
    out_shape=jax.ShapeDtypeStruct((1,), jnp.float32),
    scratch_shapes=[pltpu.SMEM((1,), jnp.float32)],
)

_NW = 32
_B_PER_W = _B // _NW
_CHUNK = 640
_NCHUNK = _B_PER_W // _CHUNK


def _gather_body(table_hbm, idx_hbm, out_hbm, idx_v, rows_v, sem):
    wid = lax.axis_index("s") * 2 + lax.axis_index("c")
    base = wid * _B_PER_W

    def chunk(c, carry):
        off = pl.multiple_of(base + c * _CHUNK, _CHUNK)
        pltpu.sync_copy(idx_hbm.at[pl.ds(off, _CHUNK)], idx_v)
        pltpu.async_copy(table_hbm.at[idx_v], rows_v, sem).wait()
        pltpu.sync_copy(rows_v, out_hbm.at[pl.ds(off, _CHUNK)])
        return carry

    lax.fori_loop(0, _NCHUNK, chunk, 0)


_gather_call = functools.partial(
    pl.kernel,
    mesh=plsc.VectorSubcoreMesh(core_axis_name="c", subcore_axis_name="s"),
    compiler_params=pltpu.CompilerParams(use_tc_tiling_on_sc=False),
    out_type=jax.ShapeDtypeStruct((_B, _EMB_DIM), jnp.float32),
    scratch_types=[
        pltpu.VMEM((_CHUNK,), jnp.int32),
        pltpu.VMEM((_CHUNK, _EMB_DIM), jnp.float32),
        pltpu.SemaphoreType.DMA,
    ],
)(_gather_body)

_QROWS = 8192


def _quant_body(scale_ref, g_ref, out_ref):
    q = jnp.round(g_ref[...] * (1.0 / scale_ref[0]))
    out_ref[...] = jnp.clip(q, -127.0, 126.0).astype(jnp.int8)


_quant_call = pl.pallas_call(
    _quant_body,
    grid=(_B // _QROWS,),
    in_specs=[
        pl.BlockSpec(memory_space=pltpu.SMEM),
        pl.BlockSpec((_QROWS, _EMB_DIM), lambda i: (i, 0)),
    ],
    out_specs=pl.BlockSpec((_QROWS, _EMB_DIM), lambda i: (i, 0)),
    out_shape=jax.ShapeDtypeStruct((_B, _EMB_DIM), jnp.int8),
)


def kernel(x, weight):
    scale = _scale_call(weight)
    gathered = _gather_call(weight, x.reshape(-1))
    q = _quant_call(scale, gathered)
    return q.reshape(x.shape[0], x.shape[1], _EMB_DIM), scale
